# baseline (speedup 1.0000x reference)
import functools

import jax
import jax.numpy as jnp
from jax import lax
from jax.experimental import pallas as pl
from jax.experimental.pallas import tpu as pltpu
from jax.experimental.pallas import tpu_sc as plsc

N = 10000
NP = 10240
E = 320000
D = 128
CH = 80
NCH = E // CH
NC = 2
NS = 16
NW = NC * NS
RPT = NP // NS
ZR = 16
DR = NP // D
CPT = NCH // NW
NBUF = 3
NIB = 4


def _segment_sum_sc(features, src1d, dst1d):
  mesh = plsc.VectorSubcoreMesh(
      core_axis_name="c", subcore_axis_name="s", num_cores=NC, num_subcores=NS)

  @functools.partial(
      pl.kernel,
      out_type=(jax.ShapeDtypeStruct((NC, NP, D), jnp.float32),
                jax.ShapeDtypeStruct((NC, NS, DR, D), jnp.float32)),
      mesh=mesh,
      compiler_params=pltpu.CompilerParams(needs_layout_passes=False),
      scratch_types=[
          pltpu.VMEM((NIB, 2, CH), jnp.int32),
          pltpu.VMEM((NBUF * CH, D), jnp.float32),
          pltpu.VMEM((ZR, D), jnp.float32),
          pltpu.VMEM((DR, D), jnp.float32),
          pltpu.VMEM_SHARED((NP, D), jnp.float32),
          pltpu.SemaphoreType.DMA,
          pltpu.SemaphoreType.DMA,
          pltpu.SemaphoreType.DMA,
          pltpu.SemaphoreType.DMA,
      ],
  )
  def sage_kernel(feat_hbm, src_hbm, dst_hbm, out_hbm, deg_hbm,
                  idxbuf, rowsbig, zbuf, degbuf,
                  aggsh, isem, gsem, ssem, zsem):
    c = lax.axis_index("c")
    s = lax.axis_index("s")
    wid = s * NC + c
    cbase = wid * CPT

    def islot(i):
      return idxbuf.at[lax.bitwise_and(i, NIB - 1)]

    def start_idx(i):
      off = (cbase + i) * CH
      pltpu.async_copy(src_hbm.at[pl.ds(off, CH)], islot(i).at[0], isem)
      pltpu.async_copy(dst_hbm.at[pl.ds(off, CH)], islot(i).at[1], isem)

    def wait_idx(i):
      off = (cbase + i) * CH
      pltpu.make_async_copy(src_hbm.at[pl.ds(off, CH)], islot(i).at[0],
                            isem).wait()
      pltpu.make_async_copy(dst_hbm.at[pl.ds(off, CH)], islot(i).at[1],
                            isem).wait()

    def hist(i):
      drow = lax.bitwise_and(i, NIB - 1)
      for k in range(CH // 16):
        v = idxbuf[drow, 1, pl.ds(k * 16, 16)]
        cnt, last = plsc.scan_count(v)
        plsc.addupdate_scatter(
            degbuf, [lax.shift_right_logical(v, 7), lax.bitwise_and(v, 127)],
            cnt.astype(jnp.float32), mask=last)

    def buf(i):
      off = pl.multiple_of(lax.rem(i, NBUF) * CH, CH)
      return rowsbig.at[pl.ds(off, CH)]

    def start_gather(i):
      pltpu.async_copy(feat_hbm.at[islot(i).at[0]], buf(i), gsem)

    def wait_gather(i):
      pltpu.make_async_copy(feat_hbm.at[islot(i).at[0]], buf(i), gsem).wait()

    def start_scatter(i):
      pltpu.async_copy(buf(i), aggsh.at[islot(i).at[1]], ssem, add=True)

    def wait_scatter(i):
      pltpu.make_async_copy(buf(i), aggsh.at[islot(i).at[1]], ssem).wait()


    zeros16 = jnp.zeros((16,), jnp.float32)

    def zfill(i, _):
      for j in range(D // 16):
        zbuf[i, pl.ds(j * 16, 16)] = zeros16
      return 0

    def dzfill(i, _):
      for j in range(D // 16):
        degbuf[i, pl.ds(j * 16, 16)] = zeros16
      return 0

    lax.fori_loop(0, ZR, zfill, 0)

    lax.fori_loop(0, DR, dzfill, 0)

    base_row = s * RPT
    for r in range(RPT // ZR):
      pltpu.async_copy(zbuf, aggsh.at[pl.ds(base_row + r * ZR, ZR)], zsem)
    for r in range(RPT // ZR):
      pltpu.make_async_copy(zbuf, aggsh.at[pl.ds(base_row + r * ZR, ZR)],
                            zsem).wait()
    plsc.subcore_barrier()

    for i in range(NIB):
      start_idx(i)
    wait_idx(0)
    start_gather(0)
    wait_idx(1)
    start_gather(1)

    wait_gather(0)
    start_scatter(0)
    hist(0)
    start_idx(NIB)
    wait_idx(2)
    start_gather(2)

    def inner(i, _):
      wait_gather(i)
      start_scatter(i)
      hist(i)

      @pl.when(i + NIB < CPT)
      def _():
        start_idx(i + NIB)

      wait_scatter(i - 1)
      wait_idx(i + 2)
      start_gather(i + 2)
      return 0

    lax.fori_loop(1, CPT - 2, inner, 0)
    for i in (CPT - 2, CPT - 1):
      wait_gather(i)
      start_scatter(i)
      hist(i)
    wait_scatter(CPT - 3)
    wait_scatter(CPT - 2)
    wait_scatter(CPT - 1)

    plsc.subcore_barrier()
    for r in range(RPT // ZR):
      row0 = base_row + r * ZR
      pltpu.async_copy(aggsh.at[pl.ds(row0, ZR)],
                       out_hbm.at[c, pl.ds(row0, ZR)], zsem)
    for r in range(RPT // ZR):
      row0 = base_row + r * ZR
      pltpu.make_async_copy(aggsh.at[pl.ds(row0, ZR)],
                            out_hbm.at[c, pl.ds(row0, ZR)], zsem).wait()
    pltpu.sync_copy(degbuf, deg_hbm.at[c, s])

  return sage_kernel(features, src1d, dst1d)


def _mlp_body(p_ref, d_ref, f_ref, a_ref, wlt, wrt, bl, w1a, w1b, b1, w2f, b2f,
              o_ref):
  R = p_ref.shape[1]
  agg = p_ref[0] + p_ref[1]
  dd = jnp.sum(d_ref[...], axis=(0, 1))
  r8 = 1.0 / jnp.maximum(dd, 1.0)
  rows8 = lax.broadcasted_iota(jnp.int32, (R, R // 128), 0) // 128
  cols8 = lax.broadcasted_iota(jnp.int32, (R, R // 128), 1)
  sel = (rows8 == cols8).astype(jnp.float32)
  z = jnp.dot(sel, r8, preferred_element_type=jnp.float32)
  lane = lax.broadcasted_iota(jnp.int32, (R, D), 1)
  row = lax.broadcasted_iota(jnp.int32, (R, D), 0)
  recip = jnp.sum(jnp.where(lane == row % 128, z, 0.0), axis=1,
                  keepdims=True)
  mean = agg * recip
  x = (jnp.dot(mean, wlt[...], preferred_element_type=jnp.float32)
       + jnp.dot(f_ref[...], wrt[...], preferred_element_type=jnp.float32)
       + bl[...])
  x = jnp.where(x >= 0, x, 0.01 * x)
  h = (jnp.dot(x, w1a[...], preferred_element_type=jnp.float32)
       + jnp.dot(a_ref[...], w1b[...], preferred_element_type=jnp.float32)
       + b1[...])
  h = jnp.maximum(h, 0.0)
  o_ref[...] = jnp.dot(h, w2f[...], preferred_element_type=jnp.float32) \
      + b2f[...]


def _mlp_tc(parts, degparts, features, additional,
            wlt, wrt, bl, w1a, w1b, b1, w2f, b2f):
  R = 1024
  grid = (NP // R,)
  full = lambda shape: pl.BlockSpec(shape, lambda i: (0,) * len(shape))
  return pl.pallas_call(
      _mlp_body,
      grid=grid,
      in_specs=[
          pl.BlockSpec((NC, R, D), lambda i: (0, i, 0)),
          pl.BlockSpec((NC, NS, R // D, D), lambda i: (0, 0, i, 0)),
          pl.BlockSpec((R, D), lambda i: (i, 0)),
          pl.BlockSpec((R, 20), lambda i: (i, 0)),
          full((D, D)), full((D, D)), full((1, D)),
          full((D, 37)), full((20, 37)), full((1, 37)),
          full((37, 3)), full((1, 3)),
      ],
      out_specs=pl.BlockSpec((R, 3), lambda i: (i, 0)),
      out_shape=jax.ShapeDtypeStruct((N, 3), jnp.float32),
  )(parts, degparts, features, additional,
    wlt, wrt, bl, w1a, w1b, b1, w2f, b2f)


def kernel(features, edges, edges2, edge_features, additional_feature,
           W_l, b_l, W_r, W1, b1, W2, b2, gamma, beta, run_mean, run_var):
  parts, degparts = _segment_sum_sc(features, edges[0], edges[1])

  scale = gamma / jnp.sqrt(run_var + 1e-5)
  shift = beta - run_mean * scale
  w2f = (W2 * scale[None, :]).T
  b2f = b2 + shift @ W2.T

  return _mlp_tc(
      parts, degparts, features, additional_feature,
      W_l.T, W_r.T, b_l[None, :],
      W1[:, :D].T, W1[:, D:].T, b1[None, :],
      w2f, b2f[None, :])


# speedup vs baseline: 1.0598x; 1.0598x over previous
import functools

import jax
import jax.numpy as jnp
from jax import lax
from jax.experimental import pallas as pl
from jax.experimental.pallas import tpu as pltpu
from jax.experimental.pallas import tpu_sc as plsc

N = 10000
NP = 10240
E = 320000
D = 128
CH = 80
NCH = E // CH
NC = 2
NS = 16
NW = NC * NS
RPT = NP // NS
ZR = 16
DR = NP // D
CPT = NCH // NW
NBUF = 3
NIB = 4


def _segment_sum_sc(features, edgv):
  mesh = plsc.VectorSubcoreMesh(
      core_axis_name="c", subcore_axis_name="s", num_cores=NC, num_subcores=NS)

  @functools.partial(
      pl.kernel,
      out_type=(jax.ShapeDtypeStruct((NC, NP, D), jnp.float32),
                jax.ShapeDtypeStruct((NC, NS, DR, D), jnp.float32)),
      mesh=mesh,
      compiler_params=pltpu.CompilerParams(needs_layout_passes=False),
      scratch_types=[
          pltpu.VMEM((NIB, 2, CH), jnp.int32),
          pltpu.VMEM((NBUF * CH, D), jnp.float32),
          pltpu.VMEM((ZR, D), jnp.float32),
          pltpu.VMEM((DR, D), jnp.float32),
          pltpu.VMEM_SHARED((NP, D), jnp.float32),
          pltpu.SemaphoreType.DMA,
          pltpu.SemaphoreType.DMA,
          pltpu.SemaphoreType.DMA,
          pltpu.SemaphoreType.DMA,
      ],
  )
  def sage_kernel(feat_hbm, edg_hbm, out_hbm, deg_hbm,
                  idxbuf, rowsbig, zbuf, degbuf,
                  aggsh, isem, gsem, ssem, zsem):
    c = lax.axis_index("c")
    s = lax.axis_index("s")
    wid = s * NC + c
    cbase = wid * CPT

    def islot(i):
      return idxbuf.at[lax.bitwise_and(i, NIB - 1)]

    def start_idx(i):
      pltpu.async_copy(edg_hbm.at[0, cbase + i], islot(i).at[0], isem)
      pltpu.async_copy(edg_hbm.at[1, cbase + i], islot(i).at[1], isem)

    def wait_idx(i):
      pltpu.make_async_copy(edg_hbm.at[0, cbase + i], islot(i).at[0],
                            isem).wait()
      pltpu.make_async_copy(edg_hbm.at[1, cbase + i], islot(i).at[1],
                            isem).wait()

    def hist(i):
      drow = lax.bitwise_and(i, NIB - 1)
      for k in range(CH // 16):
        v = idxbuf[drow, 1, pl.ds(k * 16, 16)]
        cnt, last = plsc.scan_count(v)
        plsc.addupdate_scatter(
            degbuf, [lax.shift_right_logical(v, 7), lax.bitwise_and(v, 127)],
            cnt.astype(jnp.float32), mask=last)

    def buf(i):
      off = pl.multiple_of(lax.rem(i, NBUF) * CH, CH)
      return rowsbig.at[pl.ds(off, CH)]

    def start_gather(i):
      pltpu.async_copy(feat_hbm.at[islot(i).at[0]], buf(i), gsem)

    def wait_gather(i):
      pltpu.make_async_copy(feat_hbm.at[islot(i).at[0]], buf(i), gsem).wait()

    def start_scatter(i):
      pltpu.async_copy(buf(i), aggsh.at[islot(i).at[1]], ssem, add=True)

    def wait_scatter(i):
      pltpu.make_async_copy(buf(i), aggsh.at[islot(i).at[1]], ssem).wait()


    zeros16 = jnp.zeros((16,), jnp.float32)

    def zfill(i, _):
      for j in range(D // 16):
        zbuf[i, pl.ds(j * 16, 16)] = zeros16
      return 0

    def dzfill(i, _):
      for j in range(D // 16):
        degbuf[i, pl.ds(j * 16, 16)] = zeros16
      return 0

    lax.fori_loop(0, ZR, zfill, 0)

    lax.fori_loop(0, DR, dzfill, 0)

    base_row = s * RPT
    for r in range(RPT // ZR):
      pltpu.async_copy(zbuf, aggsh.at[pl.ds(base_row + r * ZR, ZR)], zsem)
    for r in range(RPT // ZR):
      pltpu.make_async_copy(zbuf, aggsh.at[pl.ds(base_row + r * ZR, ZR)],
                            zsem).wait()
    plsc.subcore_barrier()

    for i in range(NIB):
      start_idx(i)
    wait_idx(0)
    start_gather(0)
    wait_idx(1)
    start_gather(1)

    wait_gather(0)
    start_scatter(0)
    hist(0)
    start_idx(NIB)
    wait_idx(2)
    start_gather(2)

    def inner(i, _):
      wait_gather(i)
      start_scatter(i)
      hist(i)

      @pl.when(i + NIB < CPT)
      def _():
        start_idx(i + NIB)

      wait_scatter(i - 1)
      wait_idx(i + 2)
      start_gather(i + 2)
      return 0

    lax.fori_loop(1, CPT - 2, inner, 0)
    for i in (CPT - 2, CPT - 1):
      wait_gather(i)
      start_scatter(i)
      hist(i)
    wait_scatter(CPT - 3)
    wait_scatter(CPT - 2)
    wait_scatter(CPT - 1)

    plsc.subcore_barrier()
    for r in range(RPT // ZR):
      row0 = base_row + r * ZR
      pltpu.async_copy(aggsh.at[pl.ds(row0, ZR)],
                       out_hbm.at[c, pl.ds(row0, ZR)], zsem)
    for r in range(RPT // ZR):
      row0 = base_row + r * ZR
      pltpu.make_async_copy(aggsh.at[pl.ds(row0, ZR)],
                            out_hbm.at[c, pl.ds(row0, ZR)], zsem).wait()
    pltpu.sync_copy(degbuf, deg_hbm.at[c, s])

  return sage_kernel(features, edgv)


def _mlp_body(p_ref, d_ref, f_ref, a_ref, wlt, wrt, bl, w1a, w1b, b1, w2f, b2f,
              o_ref):
  R = p_ref.shape[1]
  agg = p_ref[0] + p_ref[1]
  dd = jnp.sum(d_ref[...], axis=(0, 1))
  r8 = 1.0 / jnp.maximum(dd, 1.0)
  rows8 = lax.broadcasted_iota(jnp.int32, (R, R // 128), 0) // 128
  cols8 = lax.broadcasted_iota(jnp.int32, (R, R // 128), 1)
  sel = (rows8 == cols8).astype(jnp.float32)
  z = jnp.dot(sel, r8, preferred_element_type=jnp.float32)
  lane = lax.broadcasted_iota(jnp.int32, (R, D), 1)
  row = lax.broadcasted_iota(jnp.int32, (R, D), 0)
  recip = jnp.sum(jnp.where(lane == row % 128, z, 0.0), axis=1,
                  keepdims=True)
  mean = agg * recip
  x = (jnp.dot(mean, wlt[...], preferred_element_type=jnp.float32)
       + jnp.dot(f_ref[...], wrt[...], preferred_element_type=jnp.float32)
       + bl[...])
  x = jnp.where(x >= 0, x, 0.01 * x)
  h = (jnp.dot(x, w1a[...], preferred_element_type=jnp.float32)
       + jnp.dot(a_ref[...], w1b[...], preferred_element_type=jnp.float32)
       + b1[...])
  h = jnp.maximum(h, 0.0)
  o_ref[...] = jnp.dot(h, w2f[...], preferred_element_type=jnp.float32) \
      + b2f[...]


def _mlp_tc(parts, degparts, features, additional,
            wlt, wrt, bl, w1a, w1b, b1, w2f, b2f):
  R = 1024
  grid = (NP // R,)
  full = lambda shape: pl.BlockSpec(shape, lambda i: (0,) * len(shape))
  return pl.pallas_call(
      _mlp_body,
      grid=grid,
      in_specs=[
          pl.BlockSpec((NC, R, D), lambda i: (0, i, 0)),
          pl.BlockSpec((NC, NS, R // D, D), lambda i: (0, 0, i, 0)),
          pl.BlockSpec((R, D), lambda i: (i, 0)),
          pl.BlockSpec((R, 20), lambda i: (i, 0)),
          full((D, D)), full((D, D)), full((1, D)),
          full((D, 37)), full((20, 37)), full((1, 37)),
          full((37, 3)), full((1, 3)),
      ],
      out_specs=pl.BlockSpec((R, 3), lambda i: (i, 0)),
      out_shape=jax.ShapeDtypeStruct((N, 3), jnp.float32),
  )(parts, degparts, features, additional,
    wlt, wrt, bl, w1a, w1b, b1, w2f, b2f)


def kernel(features, edges, edges2, edge_features, additional_feature,
           W_l, b_l, W_r, W1, b1, W2, b2, gamma, beta, run_mean, run_var):
  edgv = edges.reshape(2, NCH, CH)
  parts, degparts = _segment_sum_sc(features, edgv)

  scale = gamma / jnp.sqrt(run_var + 1e-5)
  shift = beta - run_mean * scale
  w2f = (W2 * scale[None, :]).T
  b2f = b2 + shift @ W2.T

  return _mlp_tc(
      parts, degparts, features, additional_feature,
      W_l.T, W_r.T, b_l[None, :],
      W1[:, :D].T, W1[:, D:].T, b1[None, :],
      w2f, b2f[None, :])


# speedup vs baseline: 1.0747x; 1.0141x over previous
import functools

import jax
import jax.numpy as jnp
from jax import lax
from jax.experimental import pallas as pl
from jax.experimental.pallas import tpu as pltpu
from jax.experimental.pallas import tpu_sc as plsc

N = 10000
NP = 10240
E = 320000
D = 128
CH = 80
NCH = E // CH
NC = 2
NS = 16
NW = NC * NS
RPT = NP // NS
ZR = 16
DR = NP // D
CPT = NCH // NW
NBUF = 3
NIB = 4


def _segment_sum_sc(features, edgv):
  mesh = plsc.VectorSubcoreMesh(
      core_axis_name="c", subcore_axis_name="s", num_cores=NC, num_subcores=NS)

  @functools.partial(
      pl.kernel,
      out_type=(jax.ShapeDtypeStruct((NC, NP, D), jnp.float32),
                jax.ShapeDtypeStruct((NC, NS, DR, D), jnp.float32)),
      mesh=mesh,
      compiler_params=pltpu.CompilerParams(needs_layout_passes=False),
      scratch_types=[
          pltpu.VMEM((NIB, 2, CH), jnp.int32),
          pltpu.VMEM((NBUF * CH, D), jnp.float32),
          pltpu.VMEM((ZR, D), jnp.float32),
          pltpu.VMEM((DR, D), jnp.float32),
          pltpu.VMEM_SHARED((NP, D), jnp.float32),
          pltpu.SemaphoreType.DMA,
          pltpu.SemaphoreType.DMA,
          pltpu.SemaphoreType.DMA,
          pltpu.SemaphoreType.DMA,
      ],
  )
  def sage_kernel(feat_hbm, edg_hbm, out_hbm, deg_hbm,
                  idxbuf, rowsbig, zbuf, degbuf,
                  aggsh, isem, gsem, ssem, zsem):
    c = lax.axis_index("c")
    s = lax.axis_index("s")
    wid = s * NC + c
    cbase = wid * CPT

    def islot(i):
      return idxbuf.at[lax.bitwise_and(i, NIB - 1)]

    def start_idx(i):
      pltpu.async_copy(edg_hbm.at[0, cbase + i], islot(i).at[0], isem)
      pltpu.async_copy(edg_hbm.at[1, cbase + i], islot(i).at[1], isem)

    def wait_idx(i):
      pltpu.make_async_copy(edg_hbm.at[0, cbase + i], islot(i).at[0],
                            isem).wait()
      pltpu.make_async_copy(edg_hbm.at[1, cbase + i], islot(i).at[1],
                            isem).wait()

    def hist(i):
      drow = lax.bitwise_and(i, NIB - 1)
      for k in range(CH // 16):
        v = idxbuf[drow, 1, pl.ds(k * 16, 16)]
        cnt, last = plsc.scan_count(v)
        plsc.addupdate_scatter(
            degbuf, [lax.shift_right_logical(v, 7), lax.bitwise_and(v, 127)],
            cnt.astype(jnp.float32), mask=last)

    def buf(i):
      off = pl.multiple_of(lax.rem(i, NBUF) * CH, CH)
      return rowsbig.at[pl.ds(off, CH)]

    def start_gather(i):
      pltpu.async_copy(feat_hbm.at[islot(i).at[0]], buf(i), gsem)

    def wait_gather(i):
      pltpu.make_async_copy(feat_hbm.at[islot(i).at[0]], buf(i), gsem).wait()

    def start_scatter(i):
      pltpu.async_copy(buf(i), aggsh.at[islot(i).at[1]], ssem, add=True)

    def wait_scatter(i):
      pltpu.make_async_copy(buf(i), aggsh.at[islot(i).at[1]], ssem).wait()


    zeros16 = jnp.zeros((16,), jnp.float32)

    def zfill(i, _):
      for j in range(D // 16):
        zbuf[i, pl.ds(j * 16, 16)] = zeros16
      return 0

    def dzfill(i, _):
      for j in range(D // 16):
        degbuf[i, pl.ds(j * 16, 16)] = zeros16
      return 0

    lax.fori_loop(0, ZR, zfill, 0)

    lax.fori_loop(0, DR, dzfill, 0)

    base_row = s * RPT
    for r in range(RPT // ZR):
      pltpu.async_copy(zbuf, aggsh.at[pl.ds(base_row + r * ZR, ZR)], zsem)
    for r in range(RPT // ZR):
      pltpu.make_async_copy(zbuf, aggsh.at[pl.ds(base_row + r * ZR, ZR)],
                            zsem).wait()
    plsc.subcore_barrier()

    for i in range(NIB):
      start_idx(i)
    wait_idx(0)
    start_gather(0)
    wait_idx(1)
    start_gather(1)

    wait_gather(0)
    start_scatter(0)
    hist(0)
    start_idx(NIB)
    wait_idx(2)
    start_gather(2)

    def inner(i, _):
      wait_gather(i)
      start_scatter(i)
      hist(i)

      @pl.when(i + NIB < CPT)
      def _():
        start_idx(i + NIB)

      wait_scatter(i - 1)
      wait_idx(i + 2)
      start_gather(i + 2)
      return 0

    lax.fori_loop(1, CPT - 2, inner, 0)
    for i in (CPT - 2, CPT - 1):
      wait_gather(i)
      start_scatter(i)
      hist(i)
    wait_scatter(CPT - 3)
    wait_scatter(CPT - 2)
    wait_scatter(CPT - 1)

    plsc.subcore_barrier()
    for r in range(RPT // ZR):
      row0 = base_row + r * ZR
      pltpu.async_copy(aggsh.at[pl.ds(row0, ZR)],
                       out_hbm.at[c, pl.ds(row0, ZR)], zsem)
    for r in range(RPT // ZR):
      row0 = base_row + r * ZR
      pltpu.make_async_copy(aggsh.at[pl.ds(row0, ZR)],
                            out_hbm.at[c, pl.ds(row0, ZR)], zsem).wait()
    pltpu.sync_copy(degbuf, deg_hbm.at[c, s])

  return sage_kernel(features, edgv)


def _xr_body(f_ref, wrt, o_ref):
  o_ref[...] = jnp.dot(f_ref[...], wrt[...], preferred_element_type=jnp.float32)


def _xr_tc(features, wrt):
  R = 1024
  return pl.pallas_call(
      _xr_body,
      grid=(NP // R,),
      in_specs=[
          pl.BlockSpec((R, D), lambda i: (i, 0)),
          pl.BlockSpec((D, D), lambda i: (0, 0)),
      ],
      out_specs=pl.BlockSpec((R, D), lambda i: (i, 0)),
      out_shape=jax.ShapeDtypeStruct((N, D), jnp.float32),
  )(features, wrt)


def _mlp_body(p_ref, d_ref, xr_ref, a_ref, wlt, bl, w1a, w1b, b1, w2f, b2f,
              o_ref):
  R = p_ref.shape[1]
  agg = p_ref[0] + p_ref[1]
  dd = jnp.sum(d_ref[...], axis=(0, 1))
  r8 = 1.0 / jnp.maximum(dd, 1.0)
  rows8 = lax.broadcasted_iota(jnp.int32, (R, R // 128), 0) // 128
  cols8 = lax.broadcasted_iota(jnp.int32, (R, R // 128), 1)
  sel = (rows8 == cols8).astype(jnp.float32)
  z = jnp.dot(sel, r8, preferred_element_type=jnp.float32)
  lane = lax.broadcasted_iota(jnp.int32, (R, D), 1)
  row = lax.broadcasted_iota(jnp.int32, (R, D), 0)
  recip = jnp.sum(jnp.where(lane == row % 128, z, 0.0), axis=1,
                  keepdims=True)
  mean = agg * recip
  x = (jnp.dot(mean, wlt[...], preferred_element_type=jnp.float32)
       + xr_ref[...] + bl[...])
  x = jnp.where(x >= 0, x, 0.01 * x)
  h = (jnp.dot(x, w1a[...], preferred_element_type=jnp.float32)
       + jnp.dot(a_ref[...], w1b[...], preferred_element_type=jnp.float32)
       + b1[...])
  h = jnp.maximum(h, 0.0)
  o_ref[...] = jnp.dot(h, w2f[...], preferred_element_type=jnp.float32) \
      + b2f[...]


def _mlp_tc(parts, degparts, xr, additional,
            wlt, bl, w1a, w1b, b1, w2f, b2f):
  R = 2048
  grid = (NP // R,)
  full = lambda shape: pl.BlockSpec(shape, lambda i: (0,) * len(shape))
  return pl.pallas_call(
      _mlp_body,
      grid=grid,
      in_specs=[
          pl.BlockSpec((NC, R, D), lambda i: (0, i, 0)),
          pl.BlockSpec((NC, NS, R // D, D), lambda i: (0, 0, i, 0)),
          pl.BlockSpec((R, D), lambda i: (i, 0)),
          pl.BlockSpec((R, 20), lambda i: (i, 0)),
          full((D, D)), full((1, D)),
          full((D, 37)), full((20, 37)), full((1, 37)),
          full((37, 3)), full((1, 3)),
      ],
      out_specs=pl.BlockSpec((R, 3), lambda i: (i, 0)),
      out_shape=jax.ShapeDtypeStruct((N, 3), jnp.float32),
  )(parts, degparts, xr, additional,
    wlt, bl, w1a, w1b, b1, w2f, b2f)


def kernel(features, edges, edges2, edge_features, additional_feature,
           W_l, b_l, W_r, W1, b1, W2, b2, gamma, beta, run_mean, run_var):
  edgv = edges.reshape(2, NCH, CH)
  xr = _xr_tc(features, W_r.T)
  parts, degparts = _segment_sum_sc(features, edgv)

  scale = gamma / jnp.sqrt(run_var + 1e-5)
  shift = beta - run_mean * scale
  w2f = (W2 * scale[None, :]).T
  b2f = b2 + shift @ W2.T

  return _mlp_tc(
      parts, degparts, xr, additional_feature,
      W_l.T, b_l[None, :],
      W1[:, :D].T, W1[:, D:].T, b1[None, :],
      w2f, b2f[None, :])
